# baseline (speedup 1.0000x reference)
import functools
import jax
import jax.numpy as jnp
from jax import lax
from jax.experimental import pallas as pl
from jax.experimental.pallas import tpu as pltpu
from jax.experimental.pallas import tpu_sc as plsc

E = 8
D = 1024
R = 16
DQ = 1024
DV = 1024
SCALE = 32.0 / 16.0
ER = E * R
T = 2048
NW = 32
TPW = T // NW


def _stage1_kernel(h_ref, wrt_ref, qa_ref, va_ref,
                   logitsT_ref, lr_q_ref, lr_v_ref):
    h = h_ref[...]
    wr = wrt_ref[...]
    logitsT_ref[...] = lax.dot_general(
        wr, h, (((1,), (1,)), ((), ())), preferred_element_type=jnp.float32)
    lr_q_ref[...] = jnp.dot(h, qa_ref[...], preferred_element_type=jnp.float32)
    lr_v_ref[...] = jnp.dot(h, va_ref[...], preferred_element_type=jnp.float32)


def _sc_route_kernel(logitsT_hbm, scaled_hbm, idx_hbm, lt_v, sc_v, ix_v):
    wid = lax.axis_index("s") * 2 + lax.axis_index("c")
    base = wid * TPW
    for e in range(E):
        pltpu.sync_copy(logitsT_hbm.at[e, pl.ds(base, TPW)], lt_v.at[e])
    for g in range(TPW // 16):
        vecs = [lt_v[e, pl.ds(g * 16, 16)] for e in range(E)]
        m = vecs[0]
        for e in range(1, E):
            m = jnp.maximum(m, vecs[e])
        s = jnp.exp(vecs[0] - m)
        for e in range(1, E):
            s = s + jnp.exp(vecs[e] - m)
        sc_v[pl.ds(g * 16, 16)] = SCALE / s
        idx = jnp.full((16,), E - 1, dtype=jnp.int32)
        for e in range(E - 2, -1, -1):
            idx = jnp.where(vecs[e] == m, jnp.full((16,), e, jnp.int32), idx)
        ix_v[pl.ds(g * 16, 16)] = idx
    pltpu.sync_copy(sc_v, scaled_hbm.at[pl.ds(base, TPW)])
    pltpu.sync_copy(ix_v, idx_hbm.at[pl.ds(base, TPW)])


def _stage3_kernel(lr_q_ref, lr_v_ref, scaled_ref, idx_ref, qb_ref, vb_ref,
                   q_out_ref, v_out_ref):
    ts = lr_q_ref.shape[0]
    col_expert = lax.broadcasted_iota(jnp.int32, (ts, ER), 1) // R
    mask = jnp.where(col_expert == idx_ref[...], scaled_ref[...], 0.0)
    q_out_ref[...] = jnp.dot(lr_q_ref[...] * mask, qb_ref[...],
                             preferred_element_type=jnp.float32)
    v_out_ref[...] = jnp.dot(lr_v_ref[...] * mask, vb_ref[...],
                             preferred_element_type=jnp.float32)


def kernel(hidden_states, router_weight, q_lora_a, q_lora_b, v_lora_a, v_lora_b):
    orig_shape = hidden_states.shape[:-1]
    h = hidden_states.reshape(-1, D)
    qa = q_lora_a.transpose(1, 0, 2).reshape(D, ER)
    qb = q_lora_b.reshape(ER, DQ)
    va = v_lora_a.transpose(1, 0, 2).reshape(D, ER)
    vb = v_lora_b.reshape(ER, DV)

    TS = 1024
    grid = (T // TS,)
    logitsT, lr_q, lr_v = pl.pallas_call(
        _stage1_kernel,
        grid=grid,
        in_specs=[
            pl.BlockSpec((TS, D), lambda i: (i, 0)),
            pl.BlockSpec((E, D), lambda i: (0, 0)),
            pl.BlockSpec((D, ER), lambda i: (0, 0)),
            pl.BlockSpec((D, ER), lambda i: (0, 0)),
        ],
        out_specs=[
            pl.BlockSpec((E, TS), lambda i: (0, i)),
            pl.BlockSpec((TS, ER), lambda i: (i, 0)),
            pl.BlockSpec((TS, ER), lambda i: (i, 0)),
        ],
        out_shape=[
            jax.ShapeDtypeStruct((E, T), jnp.float32),
            jax.ShapeDtypeStruct((T, ER), jnp.float32),
            jax.ShapeDtypeStruct((T, ER), jnp.float32),
        ],
    )(h, router_weight, qa, va)

    sc_route = functools.partial(
        pl.kernel,
        mesh=plsc.VectorSubcoreMesh(core_axis_name="c", subcore_axis_name="s"),
        out_type=[
            jax.ShapeDtypeStruct((T,), jnp.float32),
            jax.ShapeDtypeStruct((T,), jnp.int32),
        ],
        scratch_types=[
            pltpu.VMEM((E, TPW), jnp.float32),
            pltpu.VMEM((TPW,), jnp.float32),
            pltpu.VMEM((TPW,), jnp.int32),
        ],
    )(_sc_route_kernel)
    scaled, idx = sc_route(logitsT)

    q_out, v_out = pl.pallas_call(
        _stage3_kernel,
        grid=grid,
        in_specs=[
            pl.BlockSpec((TS, ER), lambda i: (i, 0)),
            pl.BlockSpec((TS, ER), lambda i: (i, 0)),
            pl.BlockSpec((TS, 1), lambda i: (i, 0)),
            pl.BlockSpec((TS, 1), lambda i: (i, 0)),
            pl.BlockSpec((ER, DQ), lambda i: (0, 0)),
            pl.BlockSpec((ER, DV), lambda i: (0, 0)),
        ],
        out_specs=[
            pl.BlockSpec((TS, DQ), lambda i: (i, 0)),
            pl.BlockSpec((TS, DV), lambda i: (i, 0)),
        ],
        out_shape=[
            jax.ShapeDtypeStruct((T, DQ), jnp.float32),
            jax.ShapeDtypeStruct((T, DV), jnp.float32),
        ],
    )(lr_q, lr_v, scaled.reshape(T, 1), idx.reshape(T, 1), qb, vb)
    return (q_out.reshape(orig_shape + (DQ,)),
            v_out.reshape(orig_shape + (DV,)))


# speedup vs baseline: 1.7359x; 1.7359x over previous
import jax
import jax.numpy as jnp
from jax.experimental import pallas as pl
from jax.experimental.pallas import tpu as pltpu

E = 8
D = 1024
R = 16
DQ = 1024
DV = 1024
SCALE = 32.0 / 16.0
ER = E * R


def _router_lora_kernel(h_ref, wrt_ref, qa_ref, qb_ref, va_ref, vb_ref,
                        q_out_ref, v_out_ref, lrq_ref, lrv_ref):
    @pl.when(pl.program_id(1) == 0)
    def _route_and_project():
        h = h_ref[...]
        logits = jnp.dot(h, wrt_ref[...], preferred_element_type=jnp.float32)
        m = jnp.max(logits, axis=1, keepdims=True)
        score = 1.0 / jnp.sum(jnp.exp(logits - m), axis=1, keepdims=True)
        idx = jnp.argmax(logits, axis=1)
        col_expert = jax.lax.broadcasted_iota(jnp.int32, (h.shape[0], ER), 1) // R
        mask = jnp.where(col_expert == idx[:, None], score * SCALE, 0.0)
        lrq_ref[...] = jnp.dot(h, qa_ref[...], preferred_element_type=jnp.float32) * mask
        lrv_ref[...] = jnp.dot(h, va_ref[...], preferred_element_type=jnp.float32) * mask

    q_out_ref[...] = jnp.dot(lrq_ref[...], qb_ref[...],
                             preferred_element_type=jnp.float32)
    v_out_ref[...] = jnp.dot(lrv_ref[...], vb_ref[...],
                             preferred_element_type=jnp.float32)


def kernel(hidden_states, router_weight, q_lora_a, q_lora_b, v_lora_a, v_lora_b):
    orig_shape = hidden_states.shape[:-1]
    h = hidden_states.reshape(-1, D)
    T = h.shape[0]
    wrt = router_weight.T
    qa = q_lora_a.transpose(1, 0, 2).reshape(D, ER)
    qb = q_lora_b.reshape(ER, DQ)
    va = v_lora_a.transpose(1, 0, 2).reshape(D, ER)
    vb = v_lora_b.reshape(ER, DV)

    TS = 512
    CW = 512
    grid = (T // TS, DQ // CW)
    q_out, v_out = pl.pallas_call(
        _router_lora_kernel,
        grid=grid,
        in_specs=[
            pl.BlockSpec((TS, D), lambda i, j: (i, 0)),
            pl.BlockSpec((D, E), lambda i, j: (0, 0)),
            pl.BlockSpec((D, ER), lambda i, j: (0, 0)),
            pl.BlockSpec((ER, CW), lambda i, j: (0, j)),
            pl.BlockSpec((D, ER), lambda i, j: (0, 0)),
            pl.BlockSpec((ER, CW), lambda i, j: (0, j)),
        ],
        out_specs=[
            pl.BlockSpec((TS, CW), lambda i, j: (i, j)),
            pl.BlockSpec((TS, CW), lambda i, j: (i, j)),
        ],
        out_shape=[
            jax.ShapeDtypeStruct((T, DQ), jnp.float32),
            jax.ShapeDtypeStruct((T, DV), jnp.float32),
        ],
        scratch_shapes=[
            pltpu.VMEM((TS, ER), jnp.float32),
            pltpu.VMEM((TS, ER), jnp.float32),
        ],
    )(h, wrt, qa, qb, va, vb)
    return (q_out.reshape(orig_shape + (DQ,)),
            v_out.reshape(orig_shape + (DV,)))


# speedup vs baseline: 2.1277x; 1.2257x over previous
import jax
import jax.numpy as jnp
from jax.experimental import pallas as pl
from jax.experimental.pallas import tpu as pltpu

E = 8
D = 1024
R = 16
DQ = 1024
DV = 1024
SCALE = 32.0 / 16.0
ER = E * R


def _router_lora_kernel(h_ref, wrt_ref, qa_ref, qb_ref, va_ref, vb_ref,
                        q_out_ref, v_out_ref):
    h = h_ref[...]
    logits = jnp.dot(h, wrt_ref[...], preferred_element_type=jnp.float32)
    m = jnp.max(logits, axis=1, keepdims=True)
    score = 1.0 / jnp.sum(jnp.exp(logits - m), axis=1, keepdims=True)
    idx = jnp.argmax(logits, axis=1)
    col_expert = jax.lax.broadcasted_iota(jnp.int32, (h.shape[0], ER), 1) // R
    mask = jnp.where(col_expert == idx[:, None], score * SCALE, 0.0)
    lr_q = jnp.dot(h, qa_ref[...], preferred_element_type=jnp.float32) * mask
    q_out_ref[...] = jnp.dot(lr_q, qb_ref[...], preferred_element_type=jnp.float32)
    lr_v = jnp.dot(h, va_ref[...], preferred_element_type=jnp.float32) * mask
    v_out_ref[...] = jnp.dot(lr_v, vb_ref[...], preferred_element_type=jnp.float32)


def kernel(hidden_states, router_weight, q_lora_a, q_lora_b, v_lora_a, v_lora_b):
    orig_shape = hidden_states.shape[:-1]
    h = hidden_states.reshape(-1, D)
    T = h.shape[0]
    wrt = router_weight.T
    qa = q_lora_a.transpose(1, 0, 2).reshape(D, ER)
    qb = q_lora_b.reshape(ER, DQ)
    va = v_lora_a.transpose(1, 0, 2).reshape(D, ER)
    vb = v_lora_b.reshape(ER, DV)

    TS = 2048
    grid = (T // TS,)
    q_out, v_out = pl.pallas_call(
        _router_lora_kernel,
        grid=grid,
        in_specs=[
            pl.BlockSpec((TS, D), lambda i: (i, 0)),
            pl.BlockSpec((D, E), lambda i: (0, 0)),
            pl.BlockSpec((D, ER), lambda i: (0, 0)),
            pl.BlockSpec((ER, DQ), lambda i: (0, 0)),
            pl.BlockSpec((D, ER), lambda i: (0, 0)),
            pl.BlockSpec((ER, DV), lambda i: (0, 0)),
        ],
        out_specs=[
            pl.BlockSpec((TS, DQ), lambda i: (i, 0)),
            pl.BlockSpec((TS, DV), lambda i: (i, 0)),
        ],
        out_shape=[
            jax.ShapeDtypeStruct((T, DQ), jnp.float32),
            jax.ShapeDtypeStruct((T, DV), jnp.float32),
        ],
    )(h, wrt, qa, qb, va, vb)
    return (q_out.reshape(orig_shape + (DQ,)),
            v_out.reshape(orig_shape + (DV,)))


# speedup vs baseline: 2.3172x; 1.0891x over previous
import jax
import jax.numpy as jnp
from jax.experimental import pallas as pl
from jax.experimental.pallas import tpu as pltpu

E = 8
D = 1024
R = 16
DQ = 1024
DV = 1024
SCALE = 32.0 / 16.0
ER = E * R


def _router_lora_kernel(h_ref, wrt_ref, qa_ref, qb_ref, va_ref, vb_ref,
                        q_out_ref, v_out_ref):
    h = h_ref[...]
    logits = jnp.dot(h, wrt_ref[...], preferred_element_type=jnp.float32)
    m = jnp.max(logits, axis=1, keepdims=True)
    score = 1.0 / jnp.sum(jnp.exp(logits - m), axis=1, keepdims=True)
    idx = jnp.argmax(logits, axis=1)
    col_expert = jax.lax.broadcasted_iota(jnp.int32, (h.shape[0], ER), 1) // R
    mask = jnp.where(col_expert == idx[:, None], score * SCALE, 0.0)
    lr_q = jnp.dot(h, qa_ref[...], preferred_element_type=jnp.float32) * mask
    q_out_ref[...] = jnp.dot(lr_q, qb_ref[...], preferred_element_type=jnp.float32)
    lr_v = jnp.dot(h, va_ref[...], preferred_element_type=jnp.float32) * mask
    v_out_ref[...] = jnp.dot(lr_v, vb_ref[...], preferred_element_type=jnp.float32)


def kernel(hidden_states, router_weight, q_lora_a, q_lora_b, v_lora_a, v_lora_b):
    orig_shape = hidden_states.shape[:-1]
    h = hidden_states.reshape(-1, D)
    T = h.shape[0]
    wrt = router_weight.T
    qa = q_lora_a.transpose(1, 0, 2).reshape(D, ER)
    qb = q_lora_b.reshape(ER, DQ)
    va = v_lora_a.transpose(1, 0, 2).reshape(D, ER)
    vb = v_lora_b.reshape(ER, DV)

    TS = 1024
    grid = (T // TS,)
    q_out, v_out = pl.pallas_call(
        _router_lora_kernel,
        grid=grid,
        in_specs=[
            pl.BlockSpec((TS, D), lambda i: (i, 0)),
            pl.BlockSpec((D, E), lambda i: (0, 0)),
            pl.BlockSpec((D, ER), lambda i: (0, 0)),
            pl.BlockSpec((ER, DQ), lambda i: (0, 0)),
            pl.BlockSpec((D, ER), lambda i: (0, 0)),
            pl.BlockSpec((ER, DV), lambda i: (0, 0)),
        ],
        out_specs=[
            pl.BlockSpec((TS, DQ), lambda i: (i, 0)),
            pl.BlockSpec((TS, DV), lambda i: (i, 0)),
        ],
        out_shape=[
            jax.ShapeDtypeStruct((T, DQ), jnp.float32),
            jax.ShapeDtypeStruct((T, DV), jnp.float32),
        ],
    )(h, wrt, qa, qb, va, vb)
    return (q_out.reshape(orig_shape + (DQ,)),
            v_out.reshape(orig_shape + (DV,)))
